# baseline (speedup 1.0000x reference)
import functools

import jax
import jax.numpy as jnp
from jax import lax
from jax.experimental import pallas as pl
from jax.experimental.pallas import tpu as pltpu
from jax.experimental.pallas import tpu_sc as plsc

_NUM_TASKS = 3
_PROMPT_LEN = 20
_HIDDEN = 4096
_BATCH = 1024

_ROWS = _BATCH * _PROMPT_LEN
_NC = 2
_NS = 16
_L = 16
_NW = _NC * _NS
_RPW = _ROWS // _NW
_CH = 16
_NCHUNK = _RPW // _CH


def _sc_body(table_hbm, ids_hbm, out_hbm, ids_v, eidx_v, rows_v, gsem):
    wid = lax.axis_index("s") * _NC + lax.axis_index("c")
    base = wid * _RPW

    pltpu.sync_copy(ids_hbm, ids_v)

    lanes = lax.iota(jnp.int32, _L)

    def build(j, carry):
        g = base + j * _L + lanes
        b = lax.div(g, _PROMPT_LEN)
        p = g - b * _PROMPT_LEN
        t = plsc.load_gather(ids_v, [b])
        eidx_v[pl.ds(j * _L, _L)] = t * _PROMPT_LEN + p
        return carry

    lax.fori_loop(0, _RPW // _L, build, 0)

    def step(c, carry):
        r0 = pl.multiple_of(c * _CH, _CH)
        pltpu.async_copy(
            table_hbm.at[eidx_v.at[pl.ds(r0, _CH)]], rows_v, gsem
        ).wait()
        pltpu.sync_copy(rows_v, out_hbm.at[pl.ds(base + r0, _CH)])
        return carry

    lax.fori_loop(0, _NCHUNK, step, 0)


_sc_gather = functools.partial(
    pl.kernel,
    out_type=jax.ShapeDtypeStruct((_ROWS, _HIDDEN), jnp.float32),
    mesh=plsc.VectorSubcoreMesh(core_axis_name="c", subcore_axis_name="s"),
    compiler_params=pltpu.CompilerParams(needs_layout_passes=False),
    scratch_types=[
        pltpu.VMEM((_BATCH,), jnp.int32),
        pltpu.VMEM((_RPW,), jnp.int32),
        pltpu.VMEM((_CH, _HIDDEN), jnp.float32),
        pltpu.SemaphoreType.DMA,
    ],
)(_sc_body)


def kernel(task_ids, prompt_embeddings):
    ids = task_ids.astype(jnp.int32)
    table2 = prompt_embeddings.reshape(_NUM_TASKS * _PROMPT_LEN, _HIDDEN)
    out2 = _sc_gather(table2, ids)
    return out2.reshape(_BATCH, _PROMPT_LEN, _HIDDEN)


# speedup vs baseline: 2.0042x; 2.0042x over previous
import functools

import jax
import jax.numpy as jnp
from jax import lax
from jax.experimental import pallas as pl
from jax.experimental.pallas import tpu as pltpu
from jax.experimental.pallas import tpu_sc as plsc

_NUM_TASKS = 3
_PROMPT_LEN = 20
_HIDDEN = 4096
_BATCH = 1024

_NC = 2
_NS = 16
_NW = _NC * _NS
_SW = 4
_SB = _NW // _SW
_HSL = _HIDDEN // _SW
_BPW = _BATCH // _SB


def _sc_body(table_hbm, ids_hbm, out_hbm, ids_v, slice_v, sem):
    sid = lax.axis_index("s")
    cid = lax.axis_index("c")
    wid = sid * _NC + cid
    j = lax.rem(wid, _SW)
    i = lax.div(wid, _SW)
    joff = pl.multiple_of(j * _HSL, _HSL)
    gbase = i * _BPW

    pltpu.sync_copy(table_hbm.at[:, :, pl.ds(joff, _HSL)], slice_v)
    pltpu.sync_copy(ids_hbm, ids_v)

    def issue(g, carry):
        g0 = gbase + g * 16
        tvec = ids_v[pl.ds(g0, 16)]
        for k in range(16):
            t = tvec[k]
            b = g0 + k
            pltpu.async_copy(
                slice_v.at[pl.ds(t, 1)],
                out_hbm.at[pl.ds(b, 1), :, pl.ds(joff, _HSL)],
                sem,
            )
        return carry

    lax.fori_loop(0, _BPW // 16, issue, 0)

    def drain(e, carry):
        pltpu.make_async_copy(
            slice_v.at[pl.ds(0, 1)],
            out_hbm.at[pl.ds(0, 1), :, pl.ds(0, _HSL)],
            sem,
        ).wait()
        return carry

    lax.fori_loop(0, _BPW, drain, 0)


_sc_gather = functools.partial(
    pl.kernel,
    out_type=jax.ShapeDtypeStruct((_BATCH, _PROMPT_LEN, _HIDDEN), jnp.float32),
    mesh=plsc.VectorSubcoreMesh(core_axis_name="c", subcore_axis_name="s"),
    compiler_params=pltpu.CompilerParams(needs_layout_passes=False),
    scratch_types=[
        pltpu.VMEM((_BATCH,), jnp.int32),
        pltpu.VMEM((_NUM_TASKS, _PROMPT_LEN, _HSL), jnp.float32),
        pltpu.SemaphoreType.DMA,
    ],
)(_sc_body)


def kernel(task_ids, prompt_embeddings):
    ids = task_ids.astype(jnp.int32)
    return _sc_gather(prompt_embeddings, ids)
